# baseline (speedup 1.0000x reference)
import functools

import jax
import jax.numpy as jnp
from jax import lax
from jax.experimental import pallas as pl
from jax.experimental.pallas import tpu as pltpu
from jax.experimental.pallas import tpu_sc as plsc

BATCH = 16384
EMBED_DIM = 64
N_ROWS = 1000000

_NUM_CORES = 2
_NUM_SUBCORES = 16
_NUM_WORKERS = _NUM_CORES * _NUM_SUBCORES
_B_PER_W = BATCH // _NUM_WORKERS
_CHUNK = 128
_NUM_CHUNKS = _B_PER_W // _CHUNK


_T_BLK = 16384


def _transpose_body(x_ref, o_ref):
    y = x_ref[...].T
    o_ref[:, 0:EMBED_DIM] = y
    o_ref[:, EMBED_DIM:2 * EMBED_DIM] = y


def _tc_transpose(table_t):
    grid = (N_ROWS + _T_BLK - 1) // _T_BLK
    return pl.pallas_call(
        _transpose_body,
        grid=(grid,),
        in_specs=[pl.BlockSpec((EMBED_DIM, _T_BLK), lambda b: (0, b))],
        out_specs=pl.BlockSpec((_T_BLK, 2 * EMBED_DIM), lambda b: (b, 0)),
        out_shape=jax.ShapeDtypeStruct((N_ROWS, 2 * EMBED_DIM), jnp.float32),
    )(table_t)


def _gather_body(idx_hbm, t_hbm, out_hbm, idx_v, rows_v, sem):
    wid = lax.axis_index("s") * _NUM_CORES + lax.axis_index("c")
    base = wid * _B_PER_W
    pltpu.sync_copy(idx_hbm.at[pl.ds(wid * _NUM_CHUNKS, _NUM_CHUNKS)], idx_v)
    copies = []
    for j in range(_NUM_CHUNKS):
        copies.append(pltpu.async_copy(
            t_hbm.at[idx_v.at[j]],
            rows_v.at[pl.ds(j * _CHUNK, _CHUNK)],
            sem,
        ))
    for c in copies:
        c.wait()
    pltpu.sync_copy(rows_v, out_hbm.at[pl.ds(base, _B_PER_W)])


def _sc_gather(idx2d, table_b):
    mesh = plsc.VectorSubcoreMesh(core_axis_name="c", subcore_axis_name="s")
    k = functools.partial(
        pl.kernel,
        mesh=mesh,
        out_type=jax.ShapeDtypeStruct((BATCH, 2 * EMBED_DIM), jnp.float32),
        scratch_types=[
            pltpu.VMEM((_NUM_CHUNKS, _CHUNK), jnp.int32),
            pltpu.VMEM((_B_PER_W, 2 * EMBED_DIM), jnp.float32),
            pltpu.SemaphoreType.DMA,
        ],
    )(_gather_body)
    return k(idx2d, table_b)


def kernel(user, item, user_table, item_table):
    ut_b = _tc_transpose(user_table.T)
    it_b = _tc_transpose(item_table.T)
    user2d = user.reshape(BATCH // _CHUNK, _CHUNK)
    item2d = item.reshape(BATCH // _CHUNK, _CHUNK)
    ue_b = _sc_gather(user2d, ut_b)
    ie_b = _sc_gather(item2d, it_b)
    ue = ue_b[:, :EMBED_DIM]
    ie = ie_b[:, :EMBED_DIM]
    return (ue, ie)
